# baseline (speedup 1.0000x reference)
import functools

import jax
import jax.numpy as jnp
from jax import lax
from jax.experimental import pallas as pl
from jax.experimental.pallas import tpu as pltpu
from jax.experimental.pallas import tpu_sc as plsc

_DIM = 64
_LANES = 128
_N_WORKERS = 32
_VT_FULL = 7812
_V_TAIL = _VT_FULL * _LANES


def _iota16():
    return lax.iota(jnp.int32, 16)


def _transpose_kernel(wt_hbm, wtail_hbm, pair_hbm, aslab0, aslab1,
                      pslab0, pslab1, isem0, isem1, wsem0, wsem1):
    wid = lax.axis_index("s") * 2 + lax.axis_index("c")
    aslabs = (aslab0, aslab1)
    pslabs = (pslab0, pslab1)
    isems = (isem0, isem1)
    wsems = (wsem0, wsem1)
    iot = _iota16()
    dvecs = [iot + 16 * m for m in range(4)]

    def vt_of(k):
        return k * _N_WORKERS + wid

    def in_start(k, u):
        pltpu.make_async_copy(
            wt_hbm.at[:, pl.ds(vt_of(k) * _LANES, _LANES)],
            aslabs[u], isems[u]).start()

    def in_wait(u):
        pltpu.make_async_copy(
            wt_hbm.at[:, pl.ds(0, _LANES)], aslabs[u], isems[u]).wait()

    def wb_start(k, u):
        pltpu.make_async_copy(
            pslabs[u], pair_hbm.at[pl.ds(vt_of(k) * _DIM, _DIM)],
            wsems[u]).start()

    def wb_wait(u):
        pltpu.make_async_copy(
            pslabs[u], pair_hbm.at[pl.ds(0, _DIM)], wsems[u]).wait()

    def transpose(u, nrows):
        a, p = aslabs[u], pslabs[u]
        for r in range(nrows):
            for m in range(8):
                h = m // 4
                vals = plsc.load_gather(
                    a, [dvecs[m % 4], jnp.full((16,), 2 * r + h, jnp.int32)])
                p[r, pl.ds(16 * m, 16)] = vals

    @pl.when(vt_of(0) < _VT_FULL)
    def _():
        in_start(0, 0)

    @pl.when(vt_of(1) < _VT_FULL)
    def _():
        in_start(1, 1)

    def block(it, carry):
        for u in range(2):
            k = 2 * it + u

            @pl.when(vt_of(k) < _VT_FULL)
            def _():
                in_wait(u)

                @pl.when(k >= 2)
                def _():
                    wb_wait(u)
                transpose(u, _DIM)
                wb_start(k, u)

                @pl.when(vt_of(k + 2) < _VT_FULL)
                def _():
                    in_start(k + 2, u)
        return carry

    n_slots = _VT_FULL // _N_WORKERS + 1
    lax.fori_loop(0, (n_slots + 1) // 2, block, 0)
    wb_wait(0)
    wb_wait(1)

    @pl.when(wid == _N_WORKERS - 1)
    def _():
        pltpu.sync_copy(wtail_hbm, aslab0)
        transpose(0, _DIM)
        pltpu.sync_copy(pslab0, pair_hbm.at[pl.ds(_V_TAIL // 2 - 32, _DIM)])


def _gather_kernel(ids_hbm, pair_hbm, out_hbm, idx_v, idxp_v,
                   gbuf0, gbuf1, obuf0, obuf1,
                   gsem0, gsem1, wsem0, wsem1, *, rows_per_w):
    wid = lax.axis_index("s") * 2 + lax.axis_index("c")
    gbufs = (gbuf0, gbuf1)
    obufs = (obuf0, obuf1)
    gsems = (gsem0, gsem1)
    wsems = (wsem0, wsem1)
    iot = _iota16()
    bvecs = [iot + 16 * q for q in range(8)]

    pltpu.sync_copy(ids_hbm.at[:, pl.ds(wid * _LANES, _LANES)], idx_v)

    def idx_body(j, carry):
        for g in range(_LANES // 16):
            sl = pl.ds(g * 16, 16)
            idxp_v[j, sl] = lax.shift_right_logical(idx_v[j, sl], 1)
        return carry
    lax.fori_loop(0, rows_per_w, idx_body, 0)

    def gather_start(s, u):
        pltpu.make_async_copy(pair_hbm.at[idxp_v.at[s]], gbufs[u],
                              gsems[u]).start()

    def gather_wait(u):
        pltpu.make_async_copy(pair_hbm.at[idxp_v.at[0]], gbufs[u],
                              gsems[u]).wait()

    def wb_start(s, u):
        pltpu.make_async_copy(
            obufs[u], out_hbm.at[s, :, pl.ds(wid * _LANES, _LANES)],
            wsems[u]).start()

    def wb_wait(u):
        pltpu.make_async_copy(
            obufs[u], out_hbm.at[0, :, pl.ds(wid * _LANES, _LANES)],
            wsems[u]).wait()

    def merge(s, u):
        g, o = gbufs[u], obufs[u]
        for q in range(8):
            hv = (idx_v[s, pl.ds(16 * q, 16)] & 1) * _DIM
            for dd in range(_DIM):
                o[dd, pl.ds(16 * q, 16)] = plsc.load_gather(
                    g, [bvecs[q], hv + dd])

    gather_start(0, 0)
    gather_start(1, 1)

    def block(it, carry):
        for u in range(2):
            s = 2 * it + u
            gather_wait(u)

            @pl.when(s >= 2)
            def _():
                wb_wait(u)
            merge(s, u)
            wb_start(s, u)

            @pl.when(s + 2 < rows_per_w)
            def _():
                gather_start(s + 2, u)
        return carry

    lax.fori_loop(0, rows_per_w // 2, block, 0)
    wb_wait(0)
    wb_wait(1)


def kernel(input_ids, weight):
    b, s = input_ids.shape
    n, d = weight.shape
    mesh = plsc.VectorSubcoreMesh(core_axis_name="c", subcore_axis_name="s")
    cparams = pltpu.CompilerParams(use_tc_tiling_on_sc=True,
                                   needs_layout_passes=False)

    wt = weight.T
    wtail = lax.slice(wt, (0, n - 2 * d), (d, n))
    ids_t = input_ids.T.astype(jnp.int32)

    transpose_run = functools.partial(
        pl.kernel,
        mesh=mesh,
        out_type=jax.ShapeDtypeStruct((n // 2, 2 * d), jnp.float32),
        scratch_types=[
            pltpu.VMEM((d, _LANES), jnp.float32),
            pltpu.VMEM((d, _LANES), jnp.float32),
            pltpu.VMEM((d, 2 * d), jnp.float32),
            pltpu.VMEM((d, 2 * d), jnp.float32),
            pltpu.SemaphoreType.DMA,
            pltpu.SemaphoreType.DMA,
            pltpu.SemaphoreType.DMA,
            pltpu.SemaphoreType.DMA,
        ],
        compiler_params=cparams,
    )(_transpose_kernel)

    pair = transpose_run(wt, wtail)

    gather_run = functools.partial(
        pl.kernel,
        mesh=mesh,
        out_type=jax.ShapeDtypeStruct((s, d, b), jnp.float32),
        scratch_types=[
            pltpu.VMEM((s, _LANES), jnp.int32),
            pltpu.VMEM((s, _LANES), jnp.int32),
            pltpu.VMEM((_LANES, 2 * d), jnp.float32),
            pltpu.VMEM((_LANES, 2 * d), jnp.float32),
            pltpu.VMEM((d, _LANES), jnp.float32),
            pltpu.VMEM((d, _LANES), jnp.float32),
            pltpu.SemaphoreType.DMA,
            pltpu.SemaphoreType.DMA,
            pltpu.SemaphoreType.DMA,
            pltpu.SemaphoreType.DMA,
        ],
        compiler_params=cparams,
    )(functools.partial(_gather_kernel, rows_per_w=s))

    out_t = gather_run(ids_t, pair)
    return out_t.transpose(2, 0, 1)


# speedup vs baseline: 1.9098x; 1.9098x over previous
import functools

import jax
import jax.numpy as jnp
from jax import lax
from jax.experimental import pallas as pl
from jax.experimental.pallas import tpu as pltpu
from jax.experimental.pallas import tpu_sc as plsc

_DIM = 64
_LANES = 128
_N_WORKERS = 32
_VT_FULL = 7812
_V_TAIL = _VT_FULL * _LANES


def _iota16():
    return lax.iota(jnp.int32, 16)


def _transpose_kernel(wt_hbm, wtail_hbm, pair_hbm, aslab0, aslab1,
                      pslab0, pslab1, isem0, isem1, wsem0, wsem1):
    wid = lax.axis_index("s") * 2 + lax.axis_index("c")
    aslabs = (aslab0, aslab1)
    pslabs = (pslab0, pslab1)
    isems = (isem0, isem1)
    wsems = (wsem0, wsem1)
    iot = _iota16()
    dvecs = [iot + 16 * m for m in range(4)]

    def vt_of(k):
        return k * _N_WORKERS + wid

    def in_start(k, u):
        pltpu.make_async_copy(
            wt_hbm.at[:, pl.ds(vt_of(k) * _LANES, _LANES)],
            aslabs[u], isems[u]).start()

    def in_wait(u):
        pltpu.make_async_copy(
            wt_hbm.at[:, pl.ds(0, _LANES)], aslabs[u], isems[u]).wait()

    def wb_start(k, u):
        pltpu.make_async_copy(
            pslabs[u], pair_hbm.at[pl.ds(vt_of(k) * _DIM, _DIM)],
            wsems[u]).start()

    def wb_wait(u):
        pltpu.make_async_copy(
            pslabs[u], pair_hbm.at[pl.ds(0, _DIM)], wsems[u]).wait()

    def transpose(u, nrows):
        a, p = aslabs[u], pslabs[u]

        @plsc.parallel_loop(0, nrows, unroll=8)
        def _(r):
            for m in range(8):
                h = m // 4
                vals = plsc.load_gather(
                    a, [dvecs[m % 4], jnp.full((16,), 2 * r + h, jnp.int32)])
                p[r, pl.ds(16 * m, 16)] = vals

    @pl.when(vt_of(0) < _VT_FULL)
    def _():
        in_start(0, 0)

    @pl.when(vt_of(1) < _VT_FULL)
    def _():
        in_start(1, 1)

    def block(it, carry):
        for u in range(2):
            k = 2 * it + u

            @pl.when(vt_of(k) < _VT_FULL)
            def _():
                in_wait(u)

                @pl.when(k >= 2)
                def _():
                    wb_wait(u)
                transpose(u, _DIM)
                wb_start(k, u)

                @pl.when(vt_of(k + 2) < _VT_FULL)
                def _():
                    in_start(k + 2, u)
        return carry

    n_slots = _VT_FULL // _N_WORKERS + 1
    lax.fori_loop(0, (n_slots + 1) // 2, block, 0)
    wb_wait(0)
    wb_wait(1)

    @pl.when(wid == _N_WORKERS - 1)
    def _():
        pltpu.sync_copy(wtail_hbm, aslab0)
        transpose(0, _DIM)
        pltpu.sync_copy(pslab0, pair_hbm.at[pl.ds(_V_TAIL // 2 - 32, _DIM)])


def _gather_kernel(ids_hbm, pair_hbm, out_hbm, idx_v, idxp_v,
                   gbuf0, gbuf1, obuf0, obuf1,
                   gsem0, gsem1, wsem0, wsem1, *, rows_per_w):
    wid = lax.axis_index("s") * 2 + lax.axis_index("c")
    gbufs = (gbuf0, gbuf1)
    obufs = (obuf0, obuf1)
    gsems = (gsem0, gsem1)
    wsems = (wsem0, wsem1)
    iot = _iota16()
    bvecs = [iot + 16 * q for q in range(8)]

    pltpu.sync_copy(ids_hbm.at[:, pl.ds(wid * _LANES, _LANES)], idx_v)

    def idx_body(j, carry):
        for g in range(_LANES // 16):
            sl = pl.ds(g * 16, 16)
            idxp_v[j, sl] = lax.shift_right_logical(idx_v[j, sl], 1)
        return carry
    lax.fori_loop(0, rows_per_w, idx_body, 0)

    def gather_start(s, u):
        pltpu.make_async_copy(pair_hbm.at[idxp_v.at[s]], gbufs[u],
                              gsems[u]).start()

    def gather_wait(u):
        pltpu.make_async_copy(pair_hbm.at[idxp_v.at[0]], gbufs[u],
                              gsems[u]).wait()

    def wb_start(s, u):
        pltpu.make_async_copy(
            obufs[u], out_hbm.at[s, :, pl.ds(wid * _LANES, _LANES)],
            wsems[u]).start()

    def wb_wait(u):
        pltpu.make_async_copy(
            obufs[u], out_hbm.at[0, :, pl.ds(wid * _LANES, _LANES)],
            wsems[u]).wait()

    def merge(s, u):
        g, o = gbufs[u], obufs[u]
        for q in range(8):
            hv = (idx_v[s, pl.ds(16 * q, 16)] & 1) * _DIM

            @plsc.parallel_loop(0, _DIM, unroll=8)
            def _(dd):
                o[dd, pl.ds(16 * q, 16)] = plsc.load_gather(
                    g, [bvecs[q], hv + dd])

    gather_start(0, 0)
    gather_start(1, 1)

    def block(it, carry):
        for u in range(2):
            s = 2 * it + u
            gather_wait(u)

            @pl.when(s >= 2)
            def _():
                wb_wait(u)
            merge(s, u)
            wb_start(s, u)

            @pl.when(s + 2 < rows_per_w)
            def _():
                gather_start(s + 2, u)
        return carry

    lax.fori_loop(0, rows_per_w // 2, block, 0)
    wb_wait(0)
    wb_wait(1)


def kernel(input_ids, weight):
    b, s = input_ids.shape
    n, d = weight.shape
    mesh = plsc.VectorSubcoreMesh(core_axis_name="c", subcore_axis_name="s")
    cparams = pltpu.CompilerParams(use_tc_tiling_on_sc=True,
                                   needs_layout_passes=False)

    wt = weight.T
    wtail = lax.slice(wt, (0, n - 2 * d), (d, n))
    ids_t = input_ids.T.astype(jnp.int32)

    transpose_run = functools.partial(
        pl.kernel,
        mesh=mesh,
        out_type=jax.ShapeDtypeStruct((n // 2, 2 * d), jnp.float32),
        scratch_types=[
            pltpu.VMEM((d, _LANES), jnp.float32),
            pltpu.VMEM((d, _LANES), jnp.float32),
            pltpu.VMEM((d, 2 * d), jnp.float32),
            pltpu.VMEM((d, 2 * d), jnp.float32),
            pltpu.SemaphoreType.DMA,
            pltpu.SemaphoreType.DMA,
            pltpu.SemaphoreType.DMA,
            pltpu.SemaphoreType.DMA,
        ],
        compiler_params=cparams,
    )(_transpose_kernel)

    pair = transpose_run(wt, wtail)

    gather_run = functools.partial(
        pl.kernel,
        mesh=mesh,
        out_type=jax.ShapeDtypeStruct((s, d, b), jnp.float32),
        scratch_types=[
            pltpu.VMEM((s, _LANES), jnp.int32),
            pltpu.VMEM((s, _LANES), jnp.int32),
            pltpu.VMEM((_LANES, 2 * d), jnp.float32),
            pltpu.VMEM((_LANES, 2 * d), jnp.float32),
            pltpu.VMEM((d, _LANES), jnp.float32),
            pltpu.VMEM((d, _LANES), jnp.float32),
            pltpu.SemaphoreType.DMA,
            pltpu.SemaphoreType.DMA,
            pltpu.SemaphoreType.DMA,
            pltpu.SemaphoreType.DMA,
        ],
        compiler_params=cparams,
    )(functools.partial(_gather_kernel, rows_per_w=s))

    out_t = gather_run(ids_t, pair)
    return out_t.transpose(2, 0, 1)
